# Initial kernel scaffold; baseline (speedup 1.0000x reference)
import jax
import jax.numpy as jnp
from jax.experimental import pallas as pl


def kernel(token_ids, table):
    raise NotImplementedError("write your pallas kernel here")


# speedup vs baseline: 4.9463x; 4.9463x over previous
import functools

import jax
import jax.numpy as jnp
from jax import lax
from jax.experimental import pallas as pl
from jax.experimental.pallas import tpu as pltpu, tpu_sc as plsc

VOCAB = 1000000
EMBED_DIM = 32
BATCH = 16384
HIST = 200

_NC, _NS = 2, 16
_NW = _NC * _NS
_N = BATCH * HIST
_PER_W = _N // _NW
_CHUNK = 2048
_STEPS = _PER_W // _CHUNK


def _sc_gather(idx_flat, table):
    mesh = plsc.VectorSubcoreMesh(core_axis_name="c", subcore_axis_name="s")

    @functools.partial(
        pl.kernel,
        mesh=mesh,
        out_type=jax.ShapeDtypeStruct((_N, EMBED_DIM), jnp.float32),
        scratch_types=[
            pltpu.VMEM((_CHUNK,), jnp.int32),
            pltpu.VMEM((_CHUNK, EMBED_DIM), jnp.float32),
            pltpu.SemaphoreType.DMA,
        ],
        compiler_params=pltpu.CompilerParams(use_tc_tiling_on_sc=False),
    )
    def k(idx_hbm, table_hbm, out_hbm, idx_v, rows_v, sem):
        wid = lax.axis_index("s") * _NC + lax.axis_index("c")
        base = wid * _PER_W

        def body(i, carry):
            off = base + i * _CHUNK
            pltpu.sync_copy(idx_hbm.at[pl.ds(off, _CHUNK)], idx_v)
            pltpu.async_copy(table_hbm.at[idx_v], rows_v, sem).wait()
            pltpu.sync_copy(rows_v, out_hbm.at[pl.ds(off, _CHUNK)])
            return carry

        lax.fori_loop(0, _STEPS, body, 0)

    return k(idx_flat, table)


def kernel(token_ids, table):
    idx_flat = token_ids.reshape(-1).astype(jnp.int32)
    out = _sc_gather(idx_flat, table)
    return out.reshape(token_ids.shape + (table.shape[1],))


# speedup vs baseline: 5.0428x; 1.0195x over previous
import functools

import jax
import jax.numpy as jnp
from jax import lax
from jax.experimental import pallas as pl
from jax.experimental.pallas import tpu as pltpu, tpu_sc as plsc

VOCAB = 1000000
EMBED_DIM = 32
BATCH = 16384
HIST = 200

_NC, _NS = 2, 16
_NW = _NC * _NS
_N = BATCH * HIST
_PER_W = _N // _NW
_CHUNK = 1600
_STEPS = _PER_W // _CHUNK
_S2 = _STEPS // 2


def _sc_gather(idx_flat, table):
    mesh = plsc.VectorSubcoreMesh(core_axis_name="c", subcore_axis_name="s")

    @functools.partial(
        pl.kernel,
        mesh=mesh,
        out_type=jax.ShapeDtypeStruct((_N, EMBED_DIM), jnp.float32),
        scratch_types=[
            pltpu.VMEM((_CHUNK,), jnp.int32),
            pltpu.VMEM((_CHUNK,), jnp.int32),
            pltpu.VMEM((_CHUNK, EMBED_DIM), jnp.float32),
            pltpu.VMEM((_CHUNK, EMBED_DIM), jnp.float32),
            pltpu.SemaphoreType.DMA,
            pltpu.SemaphoreType.DMA,
            pltpu.SemaphoreType.DMA,
            pltpu.SemaphoreType.DMA,
            pltpu.SemaphoreType.DMA,
            pltpu.SemaphoreType.DMA,
        ],
        compiler_params=pltpu.CompilerParams(use_tc_tiling_on_sc=False),
    )
    def k(idx_hbm, table_hbm, out_hbm, idx0, idx1, rows0, rows1,
          isem0, isem1, gsem0, gsem1, osem0, osem1):
        idx_v = (idx0, idx1)
        rows_v = (rows0, rows1)
        isem = (isem0, isem1)
        gsem = (gsem0, gsem1)
        osem = (osem0, osem1)

        wid = lax.axis_index("s") * _NC + lax.axis_index("c")
        base = wid * _PER_W

        for b in range(2):
            pltpu.async_copy(idx_hbm.at[pl.ds(base + b * _CHUNK, _CHUNK)],
                             idx_v[b], isem[b])

        def body(i2, carry):
            for b in range(2):
                off = base + (i2 * 2 + b) * _CHUNK
                pltpu.make_async_copy(
                    idx_hbm.at[pl.ds(off, _CHUNK)], idx_v[b], isem[b]).wait()
                @pl.when(i2 >= 1)
                def _():
                    pltpu.make_async_copy(
                        rows_v[b], out_hbm.at[pl.ds(off, _CHUNK)], osem[b]).wait()
                pltpu.async_copy(table_hbm.at[idx_v[b]], rows_v[b], gsem[b]).wait()
                pltpu.async_copy(rows_v[b], out_hbm.at[pl.ds(off, _CHUNK)], osem[b])
                @pl.when(i2 < _S2 - 1)
                def _():
                    pltpu.async_copy(
                        idx_hbm.at[pl.ds(off + 2 * _CHUNK, _CHUNK)],
                        idx_v[b], isem[b])
            return carry

        lax.fori_loop(0, _S2, body, 0)

        for b in range(2):
            off = base + (_STEPS - 2 + b) * _CHUNK
            pltpu.make_async_copy(
                rows_v[b], out_hbm.at[pl.ds(off, _CHUNK)], osem[b]).wait()

    return k(idx_flat, table)


def kernel(token_ids, table):
    idx_flat = token_ids.reshape(-1).astype(jnp.int32)
    out = _sc_gather(idx_flat, table)
    return out.reshape(token_ids.shape + (table.shape[1],))
